# baseline (speedup 1.0000x reference)
import functools

import jax
import jax.numpy as jnp
from jax.experimental import pallas as pl

Q_OUT = 2400
V_OUT = 768
F_IN = 1200
F_HID = 256
TOTAL = 95
NUM_ANS = (2, 2, 4, 89)
COL_START = (0, 0, 2, 6)

BM = 512


def _dot(a, b):
    return jax.lax.dot_general(
        a.astype(jnp.bfloat16), b.astype(jnp.bfloat16),
        (((1,), (0,)), ((), ())),
        preferred_element_type=jnp.float32)


def _fused_kernel(iv_ref, iq_ref, qt_ref, wv_ref, bv_ref, wq_ref, bq_ref,
                  w1_refs, b1_refs, w2_refs, b2_refs, out_ref):
    xv = jnp.tanh(_dot(iv_ref[...], wv_ref[...]) + bv_ref[...])
    xq = jnp.tanh(_dot(iq_ref[...], wq_ref[...]) + bq_ref[...])
    x = jnp.tanh(xv * xq).astype(jnp.bfloat16)
    qt = qt_ref[...]
    preds = []
    for t in range(4):
        h = jnp.tanh(_dot(x, w1_refs[t][...]) + b1_refs[t][...])
        pred = _dot(h, w2_refs[t][...]) + b2_refs[t][...]
        preds.append(jnp.where(qt == t, pred, 0.0))
    out_ref[:, 0:2] = preds[0] + preds[1]
    out_ref[:, 2:6] = preds[2]
    out_ref[:, 6:TOTAL] = preds[3]


@functools.partial(jax.jit, static_argnames=())
def kernel(input_v, input_q, question_type, W_v, b_v, W_q, b_q, cls_params):
    n = input_v.shape[0]
    qt = question_type.astype(jnp.int32).reshape(n, 1)

    full = lambda shape: pl.BlockSpec(shape, lambda i: (0,) * len(shape))
    w1_specs = [full((F_IN, F_HID)) for _ in range(4)]
    b1_specs = [full((1, F_HID)) for _ in range(4)]
    w2_specs = [full((F_HID, NUM_ANS[t])) for t in range(4)]
    b2_specs = [full((1, NUM_ANS[t])) for t in range(4)]

    out = pl.pallas_call(
        _fused_kernel,
        grid=(n // BM,),
        in_specs=[
            pl.BlockSpec((BM, V_OUT), lambda i: (i, 0)),
            pl.BlockSpec((BM, Q_OUT), lambda i: (i, 0)),
            pl.BlockSpec((BM, 1), lambda i: (i, 0)),
            full((V_OUT, F_IN)),
            full((1, F_IN)),
            full((Q_OUT, F_IN)),
            full((1, F_IN)),
            w1_specs, b1_specs, w2_specs, b2_specs,
        ],
        out_specs=pl.BlockSpec((BM, TOTAL), lambda i: (i, 0)),
        out_shape=jax.ShapeDtypeStruct((n, TOTAL), jnp.float32),
    )(input_v, input_q, qt, W_v, b_v.reshape(1, F_IN), W_q,
      b_q.reshape(1, F_IN),
      [cls_params[t][0] for t in range(4)],
      [cls_params[t][1].reshape(1, F_HID) for t in range(4)],
      [cls_params[t][2] for t in range(4)],
      [cls_params[t][3].reshape(1, NUM_ANS[t]) for t in range(4)])
    return out


# speedup vs baseline: 1.0454x; 1.0454x over previous
import functools

import jax
import jax.numpy as jnp
from jax import lax
from jax.experimental import pallas as pl
from jax.experimental.pallas import tpu as pltpu

Q_OUT = 2400
V_OUT = 768
F_IN = 1200
F_HID = 256
TOTAL = 95
NUM_ANS = {0: 2, 1: 2, 2: 4, 3: 89}
IDXS = {0: [0, 1], 1: [0, 1], 2: list(range(2, 6)), 3: list(range(6, 95))}

BM = 256


def _dot(a, b):
    return jax.lax.dot_general(
        a.astype(jnp.bfloat16), b,
        (((1,), (0,)), ((), ())),
        preferred_element_type=jnp.float32)


def _fused_kernel(iv_ref, iq_ref, qt_ref, wv_ref, bv_ref, wq_ref, bq_ref,
                  w1_ref, b1_ref, w2_ref, b2_ref, out_ref):
    xv = jnp.tanh(_dot(iv_ref[...], wv_ref[...]) + bv_ref[...])
    xq = jnp.tanh(_dot(iq_ref[...], wq_ref[...]) + bq_ref[...])
    x = jnp.tanh(xv * xq)
    h = jnp.tanh(_dot(x, w1_ref[...]) + b1_ref[...])
    qt = qt_ref[...]
    blk = lax.broadcasted_iota(jnp.int32, (BM, 4 * F_HID), 1) // F_HID
    h_m = jnp.where(blk == qt, h, 0.0)
    out = _dot(h_m, w2_ref[...])
    b2 = b2_ref[...]
    for t in range(4):
        out = out + jnp.where(qt == t, 1.0, 0.0) * b2[t][None, :]
    out_ref[...] = out


@functools.partial(jax.jit, static_argnames=())
def kernel(input_v, input_q, question_type, W_v, b_v, W_q, b_q, cls_params):
    n = input_v.shape[0]
    qt = question_type.astype(jnp.int32).reshape(n, 1)

    bf = jnp.bfloat16
    W1_all = jnp.concatenate(
        [cls_params[t][0].astype(bf) for t in range(4)], axis=1)
    b1_all = jnp.concatenate([cls_params[t][1] for t in range(4)], axis=0)
    w2_cols = []
    b2_rows = []
    for t in range(4):
        W2, b2 = cls_params[t][2], cls_params[t][3]
        idx = jnp.asarray(IDXS[t], dtype=jnp.int32)
        w2_cols.append(jnp.zeros((F_HID, TOTAL), bf).at[:, idx].set(W2.astype(bf)))
        b2_rows.append(jnp.zeros((TOTAL,), jnp.float32).at[idx].set(b2))
    W2_all = jnp.concatenate(w2_cols, axis=0)
    B2_rows = jnp.stack(b2_rows + [jnp.zeros((TOTAL,), jnp.float32)] * 4)

    out = pl.pallas_call(
        _fused_kernel,
        grid=(n // BM,),
        in_specs=[
            pl.BlockSpec((BM, V_OUT), lambda i: (i, 0)),
            pl.BlockSpec((BM, Q_OUT), lambda i: (i, 0)),
            pl.BlockSpec((BM, 1), lambda i: (i, 0)),
            pl.BlockSpec((V_OUT, F_IN), lambda i: (0, 0)),
            pl.BlockSpec((1, F_IN), lambda i: (0, 0)),
            pl.BlockSpec((Q_OUT, F_IN), lambda i: (0, 0)),
            pl.BlockSpec((1, F_IN), lambda i: (0, 0)),
            pl.BlockSpec((F_IN, 4 * F_HID), lambda i: (0, 0)),
            pl.BlockSpec((1, 4 * F_HID), lambda i: (0, 0)),
            pl.BlockSpec((4 * F_HID, TOTAL), lambda i: (0, 0)),
            pl.BlockSpec((8, TOTAL), lambda i: (0, 0)),
        ],
        out_specs=pl.BlockSpec((BM, TOTAL), lambda i: (i, 0)),
        out_shape=jax.ShapeDtypeStruct((n, TOTAL), jnp.float32),
        compiler_params=pltpu.CompilerParams(
            vmem_limit_bytes=100 * 1024 * 1024),
    )(input_v, input_q, qt, W_v.astype(bf), b_v.reshape(1, F_IN),
      W_q.astype(bf), b_q.reshape(1, F_IN), W1_all,
      b1_all.reshape(1, 4 * F_HID), W2_all, B2_rows)
    return out


# speedup vs baseline: 1.5531x; 1.4857x over previous
import functools

import jax
import jax.numpy as jnp
from jax import lax
from jax.experimental import pallas as pl
from jax.experimental.pallas import tpu as pltpu

Q_OUT = 2400
V_OUT = 768
F_IN = 1200
F_HID = 256
TOTAL = 95
NUM_ANS = {0: 2, 1: 2, 2: 4, 3: 89}
IDXS = {0: [0, 1], 1: [0, 1], 2: list(range(2, 6)), 3: list(range(6, 95))}

BM = 256


def _dot(a, b):
    return jax.lax.dot_general(
        a.astype(jnp.bfloat16), b,
        (((1,), (0,)), ((), ())),
        preferred_element_type=jnp.float32)


def _dot_tlhs(a_t, b):
    return jax.lax.dot_general(
        a_t.astype(jnp.bfloat16), b,
        (((0,), (0,)), ((), ())),
        preferred_element_type=jnp.float32)


def _fused_kernel(iv_ref, iqt_ref, qt_ref, wv_ref, bv_ref, wq_ref, bq_ref,
                  w1_ref, b1_ref, w2_ref, b2_ref, out_ref):
    xv = jnp.tanh(_dot(iv_ref[...], wv_ref[...]) + bv_ref[...])
    xq = jnp.tanh(_dot_tlhs(iqt_ref[...], wq_ref[...]) + bq_ref[...])
    x = jnp.tanh(xv * xq)
    h = jnp.tanh(_dot(x, w1_ref[...]) + b1_ref[...])
    qt = qt_ref[...]
    blk = lax.broadcasted_iota(jnp.int32, (BM, 4 * F_HID), 1) // F_HID
    h_m = jnp.where(blk == qt, h, 0.0)
    out = _dot(h_m, w2_ref[...])
    b2 = b2_ref[...]
    for t in range(4):
        out = out + jnp.where(qt == t, 1.0, 0.0) * b2[t][None, :]
    out_ref[...] = out


@functools.partial(jax.jit, static_argnames=())
def kernel(input_v, input_q, question_type, W_v, b_v, W_q, b_q, cls_params):
    n = input_v.shape[0]
    qt = question_type.astype(jnp.int32).reshape(n, 1)

    bf = jnp.bfloat16
    W1_all = jnp.concatenate(
        [cls_params[t][0].astype(bf) for t in range(4)], axis=1)
    b1_all = jnp.concatenate([cls_params[t][1] for t in range(4)], axis=0)
    w2_cols = []
    b2_rows = []
    for t in range(4):
        W2, b2 = cls_params[t][2], cls_params[t][3]
        idx = jnp.asarray(IDXS[t], dtype=jnp.int32)
        w2_cols.append(jnp.zeros((F_HID, TOTAL), bf).at[:, idx].set(W2.astype(bf)))
        b2_rows.append(jnp.zeros((TOTAL,), jnp.float32).at[idx].set(b2))
    W2_all = jnp.concatenate(w2_cols, axis=0)
    B2_rows = jnp.stack(b2_rows + [jnp.zeros((TOTAL,), jnp.float32)] * 4)

    out = pl.pallas_call(
        _fused_kernel,
        grid=(n // BM,),
        in_specs=[
            pl.BlockSpec((BM, V_OUT), lambda i: (i, 0)),
            pl.BlockSpec((Q_OUT, BM), lambda i: (0, i)),
            pl.BlockSpec((BM, 1), lambda i: (i, 0)),
            pl.BlockSpec((V_OUT, F_IN), lambda i: (0, 0)),
            pl.BlockSpec((1, F_IN), lambda i: (0, 0)),
            pl.BlockSpec((Q_OUT, F_IN), lambda i: (0, 0)),
            pl.BlockSpec((1, F_IN), lambda i: (0, 0)),
            pl.BlockSpec((F_IN, 4 * F_HID), lambda i: (0, 0)),
            pl.BlockSpec((1, 4 * F_HID), lambda i: (0, 0)),
            pl.BlockSpec((4 * F_HID, TOTAL), lambda i: (0, 0)),
            pl.BlockSpec((8, TOTAL), lambda i: (0, 0)),
        ],
        out_specs=pl.BlockSpec((BM, TOTAL), lambda i: (i, 0)),
        out_shape=jax.ShapeDtypeStruct((n, TOTAL), jnp.float32),
        compiler_params=pltpu.CompilerParams(
            vmem_limit_bytes=100 * 1024 * 1024),
    )(input_v, input_q.T, qt, W_v.astype(bf), b_v.reshape(1, F_IN),
      W_q.astype(bf), b_q.reshape(1, F_IN), W1_all,
      b1_all.reshape(1, 4 * F_HID), W2_all, B2_rows)
    return out


# speedup vs baseline: 1.6549x; 1.0655x over previous
import functools

import jax
import jax.numpy as jnp
from jax import lax
from jax.experimental import pallas as pl
from jax.experimental.pallas import tpu as pltpu

Q_OUT = 2400
V_OUT = 768
F_IN = 1200
F_HID = 256
TOTAL = 95
NUM_ANS = {0: 2, 1: 2, 2: 4, 3: 89}
IDXS = {0: [0, 1], 1: [0, 1], 2: list(range(2, 6)), 3: list(range(6, 95))}

BM = 512


def _dot(a, b):
    return jax.lax.dot_general(
        a.astype(jnp.bfloat16), b,
        (((1,), (0,)), ((), ())),
        preferred_element_type=jnp.float32)


def _dot_tlhs(a_t, b):
    return jax.lax.dot_general(
        a_t.astype(jnp.bfloat16), b,
        (((0,), (0,)), ((), ())),
        preferred_element_type=jnp.float32)


def _fused_kernel(iv_ref, iqt_ref, qt_ref, wv_ref, bv_ref, wq_ref, bq_ref,
                  w1_ref, b1_ref, w2_ref, b2_ref, out_ref):
    xv = jnp.tanh(_dot(iv_ref[...], wv_ref[...]) + bv_ref[...])
    xq = jnp.tanh(_dot_tlhs(iqt_ref[...], wq_ref[...]) + bq_ref[...])
    x = jnp.tanh(xv * xq)
    h = jnp.tanh(_dot(x, w1_ref[...]) + b1_ref[...])
    qt = qt_ref[...]
    blk = lax.broadcasted_iota(jnp.int32, (BM, 4 * F_HID), 1) // F_HID
    h_m = jnp.where(blk == qt, h, 0.0)
    out = _dot(h_m, w2_ref[...])
    b2 = b2_ref[...]
    for t in range(4):
        out = out + jnp.where(qt == t, 1.0, 0.0) * b2[t][None, :]
    out_ref[...] = out


@functools.partial(jax.jit, static_argnames=())
def kernel(input_v, input_q, question_type, W_v, b_v, W_q, b_q, cls_params):
    n = input_v.shape[0]
    qt = question_type.astype(jnp.int32).reshape(n, 1)

    bf = jnp.bfloat16
    W1_all = jnp.concatenate(
        [cls_params[t][0].astype(bf) for t in range(4)], axis=1)
    b1_all = jnp.concatenate([cls_params[t][1] for t in range(4)], axis=0)
    w2_cols = []
    b2_rows = []
    for t in range(4):
        W2, b2 = cls_params[t][2], cls_params[t][3]
        idx = jnp.asarray(IDXS[t], dtype=jnp.int32)
        w2_cols.append(jnp.zeros((F_HID, TOTAL), bf).at[:, idx].set(W2.astype(bf)))
        b2_rows.append(jnp.zeros((TOTAL,), jnp.float32).at[idx].set(b2))
    W2_all = jnp.concatenate(w2_cols, axis=0)
    B2_rows = jnp.stack(b2_rows + [jnp.zeros((TOTAL,), jnp.float32)] * 4)

    out = pl.pallas_call(
        _fused_kernel,
        grid=(n // BM,),
        in_specs=[
            pl.BlockSpec((BM, V_OUT), lambda i: (i, 0)),
            pl.BlockSpec((Q_OUT, BM), lambda i: (0, i)),
            pl.BlockSpec((BM, 1), lambda i: (i, 0)),
            pl.BlockSpec((V_OUT, F_IN), lambda i: (0, 0)),
            pl.BlockSpec((1, F_IN), lambda i: (0, 0)),
            pl.BlockSpec((Q_OUT, F_IN), lambda i: (0, 0)),
            pl.BlockSpec((1, F_IN), lambda i: (0, 0)),
            pl.BlockSpec((F_IN, 4 * F_HID), lambda i: (0, 0)),
            pl.BlockSpec((1, 4 * F_HID), lambda i: (0, 0)),
            pl.BlockSpec((4 * F_HID, TOTAL), lambda i: (0, 0)),
            pl.BlockSpec((8, TOTAL), lambda i: (0, 0)),
        ],
        out_specs=pl.BlockSpec((BM, TOTAL), lambda i: (i, 0)),
        out_shape=jax.ShapeDtypeStruct((n, TOTAL), jnp.float32),
        compiler_params=pltpu.CompilerParams(
            vmem_limit_bytes=100 * 1024 * 1024),
    )(input_v, input_q.T, qt, W_v.astype(bf), b_v.reshape(1, F_IN),
      W_q.astype(bf), b_q.reshape(1, F_IN), W1_all,
      b1_all.reshape(1, 4 * F_HID), W2_all, B2_rows)
    return out


# speedup vs baseline: 1.6692x; 1.0087x over previous
import functools

import jax
import jax.numpy as jnp
from jax import lax
from jax.experimental import pallas as pl
from jax.experimental.pallas import tpu as pltpu

Q_OUT = 2400
V_OUT = 768
F_IN = 1200
F_HID = 256
TOTAL = 95
NUM_ANS = {0: 2, 1: 2, 2: 4, 3: 89}
IDXS = {0: [0, 1], 1: [0, 1], 2: list(range(2, 6)), 3: list(range(6, 95))}

BM = 1024


def _dot(a, b):
    return jax.lax.dot_general(
        a.astype(jnp.bfloat16), b,
        (((1,), (0,)), ((), ())),
        preferred_element_type=jnp.float32)


def _dot_tlhs(a_t, b):
    return jax.lax.dot_general(
        a_t.astype(jnp.bfloat16), b,
        (((0,), (0,)), ((), ())),
        preferred_element_type=jnp.float32)


def _fused_kernel(iv_ref, iqt_ref, qt_ref, wv_ref, bv_ref, wq_ref, bq_ref,
                  w1_ref, b1_ref, w2_ref, b2_ref, out_ref):
    xv = jnp.tanh(_dot(iv_ref[...], wv_ref[...]) + bv_ref[...])
    xq = jnp.tanh(_dot_tlhs(iqt_ref[...], wq_ref[...]) + bq_ref[...])
    x = jnp.tanh(xv * xq)
    h = jnp.tanh(_dot(x, w1_ref[...]) + b1_ref[...])
    qt = qt_ref[...]
    blk = lax.broadcasted_iota(jnp.int32, (BM, 4 * F_HID), 1) // F_HID
    h_m = jnp.where(blk == qt, h, 0.0)
    out = _dot(h_m, w2_ref[...])
    b2 = b2_ref[...]
    for t in range(4):
        out = out + jnp.where(qt == t, 1.0, 0.0) * b2[t][None, :]
    out_ref[...] = out


@functools.partial(jax.jit, static_argnames=())
def kernel(input_v, input_q, question_type, W_v, b_v, W_q, b_q, cls_params):
    n = input_v.shape[0]
    qt = question_type.astype(jnp.int32).reshape(n, 1)

    bf = jnp.bfloat16
    W1_all = jnp.concatenate(
        [cls_params[t][0].astype(bf) for t in range(4)], axis=1)
    b1_all = jnp.concatenate([cls_params[t][1] for t in range(4)], axis=0)
    w2_cols = []
    b2_rows = []
    for t in range(4):
        W2, b2 = cls_params[t][2], cls_params[t][3]
        idx = jnp.asarray(IDXS[t], dtype=jnp.int32)
        w2_cols.append(jnp.zeros((F_HID, TOTAL), bf).at[:, idx].set(W2.astype(bf)))
        b2_rows.append(jnp.zeros((TOTAL,), jnp.float32).at[idx].set(b2))
    W2_all = jnp.concatenate(w2_cols, axis=0)
    B2_rows = jnp.stack(b2_rows + [jnp.zeros((TOTAL,), jnp.float32)] * 4)

    out = pl.pallas_call(
        _fused_kernel,
        grid=(n // BM,),
        in_specs=[
            pl.BlockSpec((BM, V_OUT), lambda i: (i, 0)),
            pl.BlockSpec((Q_OUT, BM), lambda i: (0, i)),
            pl.BlockSpec((BM, 1), lambda i: (i, 0)),
            pl.BlockSpec((V_OUT, F_IN), lambda i: (0, 0)),
            pl.BlockSpec((1, F_IN), lambda i: (0, 0)),
            pl.BlockSpec((Q_OUT, F_IN), lambda i: (0, 0)),
            pl.BlockSpec((1, F_IN), lambda i: (0, 0)),
            pl.BlockSpec((F_IN, 4 * F_HID), lambda i: (0, 0)),
            pl.BlockSpec((1, 4 * F_HID), lambda i: (0, 0)),
            pl.BlockSpec((4 * F_HID, TOTAL), lambda i: (0, 0)),
            pl.BlockSpec((8, TOTAL), lambda i: (0, 0)),
        ],
        out_specs=pl.BlockSpec((BM, TOTAL), lambda i: (i, 0)),
        out_shape=jax.ShapeDtypeStruct((n, TOTAL), jnp.float32),
        compiler_params=pltpu.CompilerParams(
            vmem_limit_bytes=100 * 1024 * 1024),
    )(input_v, input_q.T, qt, W_v.astype(bf), b_v.reshape(1, F_IN),
      W_q.astype(bf), b_q.reshape(1, F_IN), W1_all,
      b1_all.reshape(1, 4 * F_HID), W2_all, B2_rows)
    return out
